# baseline (speedup 1.0000x reference)
import functools

import jax
import jax.numpy as jnp
from jax import lax
from jax.experimental import pallas as pl
from jax.experimental.pallas import tpu as pltpu
from jax.experimental.pallas import tpu_sc as plsc

B = 1024
S = 128
NB = 36
H = 768
V = 30522
VD = 2048
EPS = 1e-12

NC = 2
NS = 16
NW = NC * NS

N_TOK = B * S
TOK_PER_W = N_TOK // NW
GCH = 64
N_GCH = TOK_PER_W // GCH

TEXT_BLK = 1024
VIS_BLK = 512
N_VIS = B * NB


def _sc_gather_body(table_hbm, idx_hbm, out_hbm,
                    idx0, idx1, rows0, rows1, sem0, sem1):
    wid = lax.axis_index("s") * NC + lax.axis_index("c")
    base = wid * TOK_PER_W
    idx_v = (idx0, idx1)
    rows_v = (rows0, rows1)
    sems = (sem0, sem1)

    def load_and_fire(i, b):
        off = base + i * GCH
        pltpu.sync_copy(idx_hbm.at[pl.ds(off, GCH)], idx_v[b])
        pltpu.async_copy(table_hbm.at[idx_v[b]], rows_v[b], sems[b])

    load_and_fire(0, 0)
    load_and_fire(1, 1)

    def step(g, carry):
        for b in range(2):
            i = g * 2 + b
            pltpu.make_async_copy(
                table_hbm.at[idx_v[b]], rows_v[b], sems[b]).wait()
            pltpu.sync_copy(rows_v[b], out_hbm.at[pl.ds(base + i * GCH, GCH)])

            @pl.when(g < N_GCH // 2 - 1)
            def _():
                load_and_fire(i + 2, b)
        return carry

    lax.fori_loop(0, N_GCH // 2, step, 0)


def _sc_gather(table, idx_flat):
    mesh = plsc.VectorSubcoreMesh(
        core_axis_name="c", subcore_axis_name="s",
        num_cores=NC, num_subcores=NS)
    k = pl.kernel(
        _sc_gather_body,
        out_type=jax.ShapeDtypeStruct((N_TOK, H), jnp.float32),
        mesh=mesh,
        scratch_types=[
            pltpu.VMEM((GCH,), jnp.int32),
            pltpu.VMEM((GCH,), jnp.int32),
            pltpu.VMEM((GCH, H), jnp.float32),
            pltpu.VMEM((GCH, H), jnp.float32),
            pltpu.SemaphoreType.DMA,
            pltpu.SemaphoreType.DMA,
        ],
        compiler_params=pltpu.CompilerParams(use_tc_tiling_on_sc=True),
    )
    return k(table, idx_flat)


def _layer_norm_rows(x, gamma, beta):
    mean = jnp.mean(x, axis=-1, keepdims=True)
    var = jnp.mean((x - mean) ** 2, axis=-1, keepdims=True)
    return (x - mean) / jnp.sqrt(var + EPS) * gamma + beta


def _text_body(words_ref, pos_ref, tt_ref, wpos_ref, w01_ref, gamma_ref,
               beta_ref, out_ref):
    words = words_ref[...]
    pos = pos_ref[0, 0, :]
    tt = tt_ref[0, 0, :]
    onehot = (pos[:, None] ==
              lax.broadcasted_iota(jnp.int32, (TEXT_BLK, S), 1))
    posemb = jnp.dot(onehot.astype(jnp.float32), wpos_ref[...],
                     preferred_element_type=jnp.float32)
    w0 = w01_ref[0, :]
    w1 = w01_ref[1, :]
    ttemb = w0[None, :] + tt.astype(jnp.float32)[:, None] * (w1 - w0)[None, :]
    x = words + posemb + ttemb
    out_ref[...] = _layer_norm_rows(x, gamma_ref[...], beta_ref[...])


def _vis_body(img_ref, wproj_ref, row_ref, gamma_ref, beta_ref, out_ref):
    v = jnp.dot(img_ref[...], wproj_ref[...],
                preferred_element_type=jnp.float32)
    x = v + row_ref[0, :][None, :]
    out_ref[...] = _layer_norm_rows(x, gamma_ref[...], beta_ref[...])


def _text_call(words, pos3, tt3, wpos, w01, gamma, beta):
    n_blk = N_TOK // TEXT_BLK
    return pl.pallas_call(
        _text_body,
        grid=(n_blk,),
        in_specs=[
            pl.BlockSpec((TEXT_BLK, H), lambda i: (i, 0)),
            pl.BlockSpec((1, 1, TEXT_BLK), lambda i: (i, 0, 0)),
            pl.BlockSpec((1, 1, TEXT_BLK), lambda i: (i, 0, 0)),
            pl.BlockSpec((S, H), lambda i: (0, 0)),
            pl.BlockSpec((8, H), lambda i: (0, 0)),
            pl.BlockSpec((H,), lambda i: (0,)),
            pl.BlockSpec((H,), lambda i: (0,)),
        ],
        out_specs=pl.BlockSpec((TEXT_BLK, H), lambda i: (i, 0)),
        out_shape=jax.ShapeDtypeStruct((N_TOK, H), jnp.float32),
    )(words, pos3, tt3, wpos, w01, gamma, beta)


def _vis_call(img2d, wprojT, row, gamma, beta):
    n_blk = N_VIS // VIS_BLK
    return pl.pallas_call(
        _vis_body,
        grid=(n_blk,),
        in_specs=[
            pl.BlockSpec((VIS_BLK, VD), lambda i: (i, 0)),
            pl.BlockSpec((VD, H), lambda i: (0, 0)),
            pl.BlockSpec((8, H), lambda i: (0, 0)),
            pl.BlockSpec((H,), lambda i: (0,)),
            pl.BlockSpec((H,), lambda i: (0,)),
        ],
        out_specs=pl.BlockSpec((VIS_BLK, H), lambda i: (i, 0)),
        out_shape=jax.ShapeDtypeStruct((N_VIS, H), jnp.float32),
    )(img2d, wprojT, row, gamma, beta)


def kernel(token_ids, image_feat, token_type_ids, position_ids, W_word,
           W_pos, W_tt_vis, W_pos_vis, W_proj, b_proj, gamma, beta):
    idx_flat = token_ids.reshape(-1).astype(jnp.int32)
    words = _sc_gather(W_word, idx_flat)

    pos3 = position_ids.reshape(N_TOK // TEXT_BLK, 1, TEXT_BLK)
    tt3 = token_type_ids.reshape(N_TOK // TEXT_BLK, 1, TEXT_BLK)
    text = _text_call(words, pos3, tt3, W_pos[:S], W_word[:8], gamma, beta)

    vrow = (b_proj + W_tt_vis[1] + W_pos_vis[0])[None, :]
    vrow8 = jnp.broadcast_to(vrow, (8, H))
    vis = _vis_call(image_feat.reshape(N_VIS, VD), W_proj.T, vrow8,
                    gamma, beta)

    return (text.reshape(B, S, H), vis.reshape(B, NB, H))


# speedup vs baseline: 1.3013x; 1.3013x over previous
import functools

import jax
import jax.numpy as jnp
from jax import lax
from jax.experimental import pallas as pl
from jax.experimental.pallas import tpu as pltpu
from jax.experimental.pallas import tpu_sc as plsc

B = 1024
S = 128
NB = 36
H = 768
V = 30522
VD = 2048
EPS = 1e-12

NC = 2
NS = 16
NW = NC * NS

N_TOK = B * S
TOK_PER_W = N_TOK // NW
GCH = 64
N_GCH = TOK_PER_W // GCH

TEXT_BLK = 1024
VB = 16
N_VIS = B * NB


def _sc_gather_body(table_hbm, idx_hbm, out_hbm,
                    idx_all, rows0, rows1, sem0, sem1):
    wid = lax.axis_index("s") * NC + lax.axis_index("c")
    base = wid * TOK_PER_W
    rows_v = (rows0, rows1)
    sems = (sem0, sem1)

    pltpu.sync_copy(idx_hbm.at[pl.ds(base, TOK_PER_W)], idx_all)

    def fire(i, b):
        pltpu.async_copy(
            table_hbm.at[idx_all.at[pl.ds(i * GCH, GCH)]], rows_v[b], sems[b])

    fire(0, 0)
    fire(1, 1)

    def step(g, carry):
        for b in range(2):
            i = g * 2 + b
            pltpu.make_async_copy(
                table_hbm.at[idx_all.at[pl.ds(i * GCH, GCH)]], rows_v[b],
                sems[b]).wait()
            pltpu.sync_copy(rows_v[b], out_hbm.at[pl.ds(base + i * GCH, GCH)])

            @pl.when(g < N_GCH // 2 - 1)
            def _():
                fire(i + 2, b)
        return carry

    lax.fori_loop(0, N_GCH // 2, step, 0)


def _sc_gather(table, idx_flat):
    mesh = plsc.VectorSubcoreMesh(
        core_axis_name="c", subcore_axis_name="s",
        num_cores=NC, num_subcores=NS)
    k = pl.kernel(
        _sc_gather_body,
        out_type=jax.ShapeDtypeStruct((N_TOK, H), jnp.float32),
        mesh=mesh,
        scratch_types=[
            pltpu.VMEM((TOK_PER_W,), jnp.int32),
            pltpu.VMEM((GCH, H), jnp.float32),
            pltpu.VMEM((GCH, H), jnp.float32),
            pltpu.SemaphoreType.DMA,
            pltpu.SemaphoreType.DMA,
        ],
        compiler_params=pltpu.CompilerParams(use_tc_tiling_on_sc=True),
    )
    return k(table, idx_flat)


def _layer_norm_rows(x, gamma, beta):
    mean = jnp.mean(x, axis=-1, keepdims=True)
    var = jnp.mean((x - mean) ** 2, axis=-1, keepdims=True)
    return (x - mean) / jnp.sqrt(var + EPS) * gamma + beta


def _text_body(words_ref, pos_ref, tt_ref, wpos_ref, w01_ref, gamma_ref,
               beta_ref, out_ref):
    words = words_ref[...]
    pos = pos_ref[0, 0, :]
    tt = tt_ref[0, 0, :]
    onehot = (pos[:, None] ==
              lax.broadcasted_iota(jnp.int32, (TEXT_BLK, S), 1))
    posemb = jnp.dot(onehot.astype(jnp.float32), wpos_ref[...],
                     preferred_element_type=jnp.float32)
    w0 = w01_ref[0, :]
    w1 = w01_ref[1, :]
    ttemb = w0[None, :] + tt.astype(jnp.float32)[:, None] * (w1 - w0)[None, :]
    x = words + posemb + ttemb
    out_ref[...] = _layer_norm_rows(x, gamma_ref[...], beta_ref[...])


def _vis_body(img_ref, wproj_ref, row_ref, gamma_ref, beta_ref, out_ref):
    x3 = img_ref[...]
    xm = x3.reshape(VB * NB, VD)
    v = lax.dot_general(xm, wproj_ref[...],
                        dimension_numbers=(((1,), (1,)), ((), ())),
                        preferred_element_type=jnp.float32)
    x = v + row_ref[0, :][None, :]
    y = _layer_norm_rows(x, gamma_ref[...], beta_ref[...])
    out_ref[...] = y.reshape(VB, NB, H)


def _text_call(words, pos3, tt3, wpos, w01, gamma, beta):
    n_blk = N_TOK // TEXT_BLK
    return pl.pallas_call(
        _text_body,
        grid=(n_blk,),
        in_specs=[
            pl.BlockSpec((TEXT_BLK, H), lambda i: (i, 0)),
            pl.BlockSpec((1, 1, TEXT_BLK), lambda i: (i, 0, 0)),
            pl.BlockSpec((1, 1, TEXT_BLK), lambda i: (i, 0, 0)),
            pl.BlockSpec((S, H), lambda i: (0, 0)),
            pl.BlockSpec((8, H), lambda i: (0, 0)),
            pl.BlockSpec((H,), lambda i: (0,)),
            pl.BlockSpec((H,), lambda i: (0,)),
        ],
        out_specs=pl.BlockSpec((TEXT_BLK, H), lambda i: (i, 0)),
        out_shape=jax.ShapeDtypeStruct((N_TOK, H), jnp.float32),
    )(words, pos3, tt3, wpos, w01, gamma, beta)


def _vis_call(img3d, wproj, row, gamma, beta):
    n_blk = B // VB
    return pl.pallas_call(
        _vis_body,
        grid=(n_blk,),
        in_specs=[
            pl.BlockSpec((VB, NB, VD), lambda i: (i, 0, 0)),
            pl.BlockSpec((H, VD), lambda i: (0, 0)),
            pl.BlockSpec((8, H), lambda i: (0, 0)),
            pl.BlockSpec((H,), lambda i: (0,)),
            pl.BlockSpec((H,), lambda i: (0,)),
        ],
        out_specs=pl.BlockSpec((VB, NB, H), lambda i: (i, 0, 0)),
        out_shape=jax.ShapeDtypeStruct((B, NB, H), jnp.float32),
    )(img3d, wproj, row, gamma, beta)


def kernel(token_ids, image_feat, token_type_ids, position_ids, W_word,
           W_pos, W_tt_vis, W_pos_vis, W_proj, b_proj, gamma, beta):
    idx_flat = token_ids.reshape(-1)
    words = _sc_gather(W_word, idx_flat)

    pos3 = position_ids.reshape(N_TOK // TEXT_BLK, 1, TEXT_BLK)
    tt3 = token_type_ids.reshape(N_TOK // TEXT_BLK, 1, TEXT_BLK)
    text = _text_call(words, pos3, tt3, W_pos[:S], W_word[:8], gamma, beta)

    vrow = (b_proj + W_tt_vis[1] + W_pos_vis[0])[None, :]
    vrow8 = jnp.broadcast_to(vrow, (8, H))
    vis = _vis_call(image_feat, W_proj, vrow8, gamma, beta)

    return (text.reshape(B, S, H), vis)
